# baseline (speedup 1.0000x reference)
import functools

import jax
import jax.numpy as jnp
from jax import lax
from jax.experimental import pallas as pl
from jax.experimental.pallas import tpu as pltpu
from jax.experimental.pallas import tpu_sc as plsc

_B, _T, _NPT = 4, 24, 2000
_N = _T * _NPT
_E = 339072
_F, _H, _TGT = 16, 64, 4
_LH = 128
_RH = 256

_C = 128
_EP = 339968
_NB = _N // 16
_ECT = _EP // 16
_ZR = 600
_WC = 32
_NCK = 8


@functools.cache
def _make_hop():
    nch = _ECT // _C
    grp = _WC // 16

    @functools.partial(
        pl.kernel,
        out_type=jax.ShapeDtypeStruct((_NCK * _N, _WC), jnp.float32),
        mesh=plsc.VectorSubcoreMesh(core_axis_name="c", subcore_axis_name="s"),
        compiler_params=pltpu.CompilerParams(use_tc_tiling_on_sc=False),
        scratch_types=[
            pltpu.VMEM((_C,), jnp.int32),
            pltpu.VMEM((_C,), jnp.int32),
            pltpu.VMEM((_C, _WC), jnp.float32),
            pltpu.VMEM((_ZR, _WC), jnp.float32),
            pltpu.VMEM((_C, _WC), jnp.float32),
            pltpu.VMEM((16,), jnp.int32),
            pltpu.VMEM_SHARED((_N, _WC), jnp.float32),
            pltpu.SemaphoreType.DMA,
        ],
    )
    def hop(x_hbm, srcoff_hbm, dst_hbm, ew_hbm, np_hbm, out_hbm,
            src_v, dst_v, rows_v, zbuf, ewrow_v, np_v, acc, sem):
        cid = lax.axis_index("c")
        sid = lax.axis_index("s")
        pltpu.sync_copy(np_hbm, np_v)
        npv = np_v[...][0]

        def zrow(r, _):
            for j in range(grp):
                zbuf[r, pl.ds(j * 16, 16)] = jnp.zeros((16,), jnp.float32)
            return 0

        lax.fori_loop(0, _ZR, zrow, 0)

        def pass_body(p, _):
            chunk = p * 2 + cid
            for q in range(_NB // _ZR):
                pltpu.sync_copy(zbuf,
                                acc.at[pl.ds(sid * _NB + q * _ZR, _ZR), :])
            plsc.subcore_barrier()

            def chunk_body(i, _):
                base = sid * _ECT + i * _C
                pltpu.sync_copy(srcoff_hbm.at[pl.ds(chunk * _EP + base, _C)],
                                src_v)
                pltpu.sync_copy(dst_hbm.at[pl.ds(base, _C)], dst_v)
                pltpu.sync_copy(ew_hbm.at[pl.ds(base, _C), :], ewrow_v)
                pltpu.async_copy(x_hbm.at[src_v], rows_v, sem).wait()

                def edge_body(e, _):
                    for j in range(grp):
                        rows_v[e, pl.ds(j * 16, 16)] = (
                            rows_v[e, pl.ds(j * 16, 16)]
                            * ewrow_v[e, pl.ds(j * 16, 16)])
                    return 0

                lax.fori_loop(0, _C, edge_body, 0, unroll=8)
                pltpu.sync_copy(rows_v, acc.at[dst_v], add=True)
                return 0

            lax.fori_loop(0, nch, chunk_body, 0)
            plsc.subcore_barrier()
            pltpu.sync_copy(
                acc.at[pl.ds(sid * _NB, _NB), :],
                out_hbm.at[pl.ds(chunk * _N + sid * _NB, _NB), :])
            plsc.subcore_barrier()
            return 0

        lax.fori_loop(0, npv, pass_body, 0)

    return hop


def _hop(x8, srcoff, dst, ew, npass):
    return _make_hop()(x8, srcoff, dst, ew,
                       jnp.full((16,), npass, jnp.int32))


_PBLK = 2048


def _prep_body(ea_ref, ewf_ref):
    i = pl.program_id(0)
    ea = ea_ref[...]
    w = jnp.sqrt(ea[:, 0:1] ** 2 + ea[:, 1:2] ** 2 + ea[:, 2:3] ** 2)
    rid = lax.broadcasted_iota(jnp.int32, (_PBLK, 1), 0) + i * _PBLK
    w = jnp.where(rid < _E, w, 0.0)
    ewf_ref[...] = jnp.broadcast_to(w, (_PBLK, _WC))


def _prep(edge_attr):
    return pl.pallas_call(
        _prep_body,
        grid=(_EP // _PBLK,),
        in_specs=[pl.BlockSpec((_PBLK, 3), lambda i: (i, 0))],
        out_specs=pl.BlockSpec((_PBLK, _WC), lambda i: (i, 0)),
        out_shape=jax.ShapeDtypeStruct((_EP, _WC), jnp.float32),
    )(edge_attr)


_RB = 3000


def _dis_body(degm_ref, dis_ref, dis2_ref):
    d = degm_ref[...][:, 0:1]
    good = d > 0
    r = lax.rsqrt(jnp.where(good, d, 1.0))
    dis = jnp.where(good, r, 0.0)
    dis16 = jnp.broadcast_to(dis, (_RB, 16))
    dis_ref[...] = dis16
    dis2_ref[...] = dis16 * dis16


def _dis(degm):
    return pl.pallas_call(
        _dis_body,
        grid=(_N // _RB,),
        in_specs=[pl.BlockSpec((_RB, _WC), lambda i: (i, 0))],
        out_specs=[pl.BlockSpec((_RB, 16), lambda i: (i, 0))] * 2,
        out_shape=[jax.ShapeDtypeStruct((_N, 16), jnp.float32)] * 2,
    )(degm.reshape(_NCK * _N, _WC)[: _N])


_RS = 600


def _scale_call(nc, body, n_m, in_full=True):
    n_s = {_scale_body: 1, _dual_body: 2, _mid_body: 2, _post_body: 1}[body]
    n_out = 2 if body is _dual_body else 1
    in_ck = _NCK if in_full else nc

    def run(*arrays):
        outs = pl.pallas_call(
            body,
            grid=(_N // _RS,),
            in_specs=[pl.BlockSpec((nc, _RS, _WC), lambda i: (0, i, 0))] *
                     n_m + [pl.BlockSpec((_RS, 16), lambda i: (i, 0))] * n_s,
            out_specs=[pl.BlockSpec((nc, _RS, _WC),
                                    lambda i: (0, i, 0))] * n_out,
            out_shape=[jax.ShapeDtypeStruct((_NCK, _N, _WC),
                                            jnp.float32)] * n_out,
        )(*([a.reshape(in_ck, _N, _WC) for a in arrays[:n_m]]
            + list(arrays[n_m:])))
        res = [o.reshape(_NCK * _N, _WC) for o in outs]
        return res[0] if n_out == 1 else res
    return run


def _scale_body(m, s, o):
    o[...] = m[...] * s[...][:, 0:1][None]


def _dual_body(m, s1, s2, o1, o2):
    mv = m[...]
    o1[...] = mv * s1[...][:, 0:1][None]
    o2[...] = mv * s2[...][:, 0:1][None]


def _mid_body(m, init, s2, s1, o):
    o[...] = (m[...] * s2[...][:, 0:1][None]
              + init[...] * s1[...][:, 0:1][None])


def _post_body(m, init, s1, o):
    o[...] = m[...] * s1[...][:, 0:1][None] + init[...]


_R = 480


def _l1_body(h0, h1, h2, h3, w1, w2, b1, dis, h1f, z0, z1, z2, z3):
    hks = [h0, h1, h2, h3]
    zrefs = [z0, z1, z2, z3]
    dcol = dis[...][:, 0:1]
    for b in range(_B):
        c, o = b // 2, (b % 2) * 16
        xb = jnp.concatenate([hk[c, :, o:o + 16] for hk in hks], axis=1)
        pre = jnp.dot(xb, w1[...], preferred_element_type=jnp.float32)
        h1b = jnp.maximum(pre + b1[...], 0.0)
        z2b = jnp.dot(h1b, w2[...], preferred_element_type=jnp.float32)
        h1f[2 * b] = h1b[:, :32]
        h1f[2 * b + 1] = h1b[:, 32:]
        for k in range(4):
            zk = z2b[:, 64 * k:64 * k + 64]
            if k == 3:
                zk = zk * dcol
            zrefs[k][2 * b] = zk[:, :32]
            zrefs[k][2 * b + 1] = zk[:, 32:]


def _l1(hk_list, w1r, w2c, b1r, dis16):
    big = jax.ShapeDtypeStruct((_NCK, _N, _WC), jnp.float32)
    outs = pl.pallas_call(
        _l1_body,
        grid=(_N // _R,),
        in_specs=[pl.BlockSpec((2, _R, _WC), lambda i: (0, i, 0))] * 4 + [
            pl.BlockSpec((64, 64), lambda i: (0, 0)),
            pl.BlockSpec((64, 256), lambda i: (0, 0)),
            pl.BlockSpec((1, 64), lambda i: (0, 0)),
            pl.BlockSpec((_R, 16), lambda i: (i, 0)),
        ],
        out_specs=[pl.BlockSpec((_NCK, _R, _WC), lambda i: (0, i, 0))] * 5,
        out_shape=[big] * 5,
    )(*[h.reshape(-1, _N, _WC)[:2] for h in hk_list], w1r, w2c, b1r, dis16)
    return [o.reshape(_NCK * _N, _WC) for o in outs]


def _l2_body(o2, h1f, w3, b2, dis, z0, z1, z2, z3):
    zrefs = [z0, z1, z2, z3]
    dcol = dis[...][:, 0:1]
    parts = [[] for _ in range(4)]
    for b in range(_B):
        o2b = jnp.concatenate([o2[2 * b], o2[2 * b + 1]], axis=1)
        h1b = jnp.concatenate([h1f[2 * b], h1f[2 * b + 1]], axis=1)
        h2b = (jnp.maximum(o2b + b2[...], 0.0) + h1b) * 0.5
        z3b = jnp.dot(h2b, w3[...], preferred_element_type=jnp.float32)
        for k in range(4):
            parts[k].append(z3b[:, 4 * k:4 * k + 4])
    zeros16 = jnp.zeros((_R, 16), jnp.float32)
    for k in range(4):
        zk = jnp.concatenate(parts[k], axis=1)
        if k == 3:
            zk = zk * dcol
        zrefs[k][0] = jnp.concatenate([zk, zeros16], axis=1)
        zrefs[k][1] = jnp.zeros((_R, _WC), jnp.float32)


def _l2(out2f, h1f, w3c, b2r, dis16):
    small = jax.ShapeDtypeStruct((_NCK, _N, _WC), jnp.float32)
    outs = pl.pallas_call(
        _l2_body,
        grid=(_N // _R,),
        in_specs=[pl.BlockSpec((8, _R, _WC), lambda i: (0, i, 0))] * 2 + [
            pl.BlockSpec((64, 16), lambda i: (0, 0)),
            pl.BlockSpec((1, 64), lambda i: (0, 0)),
            pl.BlockSpec((_R, 16), lambda i: (i, 0)),
        ],
        out_specs=[pl.BlockSpec((2, _R, _WC), lambda i: (0, i, 0))] * 4,
        out_shape=[small] * 4,
    )(out2f.reshape(_NCK, _N, _WC), h1f.reshape(_NCK, _N, _WC),
      w3c, b2r, dis16)
    return [o.reshape(_NCK * _N, _WC) for o in outs]


def _tail_body(lstm_in_ref, b3f_ref, Wih0_ref, Whh0_ref, b0_ref, Wih1_ref,
               Whh1_ref, b1_ref, Wr1_ref, br1_ref, Wr2_ref, br2_ref, Wf_ref,
               bf_ref, We_ref, be_ref, o_ref, eo_ref):
    IN0 = _TGT * _NPT
    xin = lstm_in_ref[...].reshape(_B * _T, IN0)
    xin = jnp.maximum(xin + b3f_ref[...], 0.0)
    xp0 = jnp.dot(xin, Wih0_ref[...], preferred_element_type=jnp.float32)
    xp0 = xp0 + b0_ref[...]
    xp0 = xp0.reshape(_B, _T, 4 * _LH)

    def lstm_cell(gates, c):
        i = jax.nn.sigmoid(gates[:, :_LH])
        f = jax.nn.sigmoid(gates[:, _LH:2 * _LH])
        g = jnp.tanh(gates[:, 2 * _LH:3 * _LH])
        o = jax.nn.sigmoid(gates[:, 3 * _LH:])
        c = f * c + i * g
        h = o * jnp.tanh(c)
        return h, c

    z = jnp.zeros((_B, _LH), jnp.float32)
    h0, c0, h1, c1 = z, z, z, z
    for t in range(_T):
        g0 = xp0[:, t, :] + jnp.dot(h0, Whh0_ref[...],
                                    preferred_element_type=jnp.float32)
        h0, c0 = lstm_cell(g0, c0)
        g1 = (jnp.dot(h0, Wih1_ref[...], preferred_element_type=jnp.float32)
              + jnp.dot(h1, Whh1_ref[...], preferred_element_type=jnp.float32)
              + b1_ref[...])
        h1, c1 = lstm_cell(g1, c1)

    last = jnp.maximum(h1, 0.0)
    r = jnp.maximum(jnp.dot(last, Wr1_ref[...],
                            preferred_element_type=jnp.float32)
                    + br1_ref[...], 0.0)
    r = jnp.dot(r, Wr2_ref[...], preferred_element_type=jnp.float32) + br2_ref[...]
    o_ref[...] = jnp.dot(r, Wf_ref[...],
                         preferred_element_type=jnp.float32) + bf_ref[...]
    eo_ref[...] = jnp.dot(r, We_ref[...],
                          preferred_element_type=jnp.float32) + be_ref[...]


def _tail(lstm_in, b3f, Wih0, Whh0, b0, Wih1, Whh1, b1, Wr1, br1, Wr2, br2,
          Wf, bf, We, be):
    EO = We.shape[1]
    o, eo = pl.pallas_call(
        _tail_body,
        out_shape=[
            jax.ShapeDtypeStruct((_B, 2 * _NPT), jnp.float32),
            jax.ShapeDtypeStruct((_B, EO), jnp.float32),
        ],
    )(lstm_in, b3f, Wih0, Whh0, b0, Wih1, Whh1, b1, Wr1, br1, Wr2, br2,
      Wf, bf, We, be)
    return o, eo


def kernel(x, edge_index, edge_attr, W1, b1, W2, b2, W3, b3, Wih0, Whh0,
           bih0, bhh0, Wih1, Whh1, bih1, bhh1, Wr1, br1, Wr2, br2, Wf1, bf1,
           Wf2, bf2, We, be):
    src = edge_index[0].astype(jnp.int32)
    dst = edge_index[1].astype(jnp.int32)
    pad = _EP - _E
    src_p = jnp.concatenate([src, jnp.zeros((pad,), jnp.int32)])
    dst_p = jnp.concatenate([dst, jnp.zeros((pad,), jnp.int32)])
    off8 = (jnp.arange(_NCK, dtype=jnp.int32) * _N)[:, None]
    srcoff = (src_p[None, :] + off8).reshape(-1)

    ewf = _prep(edge_attr)
    ones8 = jnp.ones((_NCK * _N, _WC), jnp.float32)
    degm = _hop(ones8, srcoff, dst_p, ewf, 1)
    dis16, dis216 = _dis(degm)

    x2 = (x.transpose(1, 0, 2).reshape(_N, _B * _F)
          .reshape(_N, 2, _WC).transpose(1, 0, 2).reshape(2 * _N, _WC))
    scale2 = _scale_call(2, _scale_body, 1, in_full=False)
    dual2 = _scale_call(2, _dual_body, 1)
    mid2 = _scale_call(2, _mid_body, 2)
    post2 = _scale_call(2, _post_body, 2)

    xs = scale2(x2, dis16)
    m = _hop(xs, srcoff, dst_p, ewf, 1)
    ha, has = dual2(m, dis16, dis216)
    m = _hop(has, srcoff, dst_p, ewf, 1)
    hb, hbs = dual2(m, dis16, dis216)
    m = _hop(hbs, srcoff, dst_p, ewf, 1)
    hc = _scale_call(2, _scale_body, 1)(m, dis16)

    w1r = W1.reshape(4 * _F, _H)
    w2c = W2.transpose(1, 0, 2).reshape(_H, 4 * _H)
    b1r = b1.reshape(1, _H)
    h1f, z2_0, z2_1, z2_2, z2_3s = _l1([x2, ha, hb, hc], w1r, w2c, b1r, dis16)

    mid8 = _scale_call(8, _mid_body, 2)
    post8 = _scale_call(8, _post_body, 2)
    m = _hop(z2_3s, srcoff, dst_p, ewf, 4)
    t = mid8(m, z2_2, dis216, dis16)
    m = _hop(t, srcoff, dst_p, ewf, 4)
    t = mid8(m, z2_1, dis216, dis16)
    m = _hop(t, srcoff, dst_p, ewf, 4)
    out2 = post8(m, z2_0, dis16)

    w3c = W3.transpose(1, 0, 2).reshape(_H, 4 * _TGT)
    b2r = b2.reshape(1, _H)
    z3_0, z3_1, z3_2, z3_3s = _l2(out2, h1f, w3c, b2r, dis16)

    m = _hop(z3_3s, srcoff, dst_p, ewf, 1)
    t = mid2(m, z3_2, dis216, dis16)
    m = _hop(t, srcoff, dst_p, ewf, 1)
    t = mid2(m, z3_1, dis216, dis16)
    m = _hop(t, srcoff, dst_p, ewf, 1)
    out3 = post2(m, z3_0, dis16)

    lstm_in = (out3[:_N, :16].reshape(_T, _NPT, _B, _TGT)
               .transpose(2, 0, 1, 3).reshape(_B, _T, _NPT * _TGT))
    b3f = jnp.tile(b3, _NPT).reshape(1, _NPT * _TGT)

    Wf = jnp.concatenate([Wf1, Wf2], axis=1)
    bf = jnp.concatenate([bf1, bf2], axis=0)
    o, eo = _tail(lstm_in, b3f, Wih0, Whh0, bih0 + bhh0, Wih1, Whh1,
                  bih1 + bhh1, Wr1, br1, Wr2, br2, Wf, bf, We, be)
    o = o.reshape(_B, 2, _NPT)
    eo = eo.reshape(_B, 2, 7064)
    return o, eo


# speedup vs baseline: 1.3766x; 1.3766x over previous
import functools

import jax
import jax.numpy as jnp
from jax import lax
from jax.experimental import pallas as pl
from jax.experimental.pallas import tpu as pltpu
from jax.experimental.pallas import tpu_sc as plsc

_B, _T, _NPT = 4, 24, 2000
_N = _T * _NPT
_E = 339072
_F, _H, _TGT = 16, 64, 4
_LH = 128
_RH = 256

_C = 128
_EP = 339968
_NB = _N // 16
_ECT = _EP // 16
_ZR = 300
_WC = 32
_NCK = 8


@functools.cache
def _make_hop():
    nch = _ECT // _C
    grp = _WC // 16

    @functools.partial(
        pl.kernel,
        out_type=jax.ShapeDtypeStruct((_NCK * _N, _WC), jnp.float32),
        mesh=plsc.VectorSubcoreMesh(core_axis_name="c", subcore_axis_name="s"),
        compiler_params=pltpu.CompilerParams(use_tc_tiling_on_sc=False),
        scratch_types=[
            pltpu.VMEM((_C,), jnp.int32),
            pltpu.VMEM((_C,), jnp.int32),
            pltpu.VMEM((_C, _WC), jnp.float32),
            pltpu.VMEM((_ZR, _WC), jnp.float32),
            pltpu.VMEM((_C, _WC), jnp.float32),
            pltpu.VMEM((16,), jnp.int32),
            pltpu.VMEM_SHARED((_N, _WC), jnp.float32),
            pltpu.SemaphoreType.DMA,
        ],
    )
    def hop(x_hbm, srcoff_hbm, dst_hbm, ew_hbm, np_hbm, out_hbm,
            src_v, dst_v, rows_v, zbuf, ewrow_v, np_v, acc, sem):
        cid = lax.axis_index("c")
        sid = lax.axis_index("s")
        pltpu.sync_copy(np_hbm, np_v)
        npv = np_v[...][0]

        def zrow(r, _):
            for j in range(grp):
                zbuf[r, pl.ds(j * 16, 16)] = jnp.zeros((16,), jnp.float32)
            return 0

        lax.fori_loop(0, _ZR, zrow, 0)

        def pass_body(p, _):
            chunk = p * 2 + cid
            for q in range(_NB // _ZR):
                pltpu.sync_copy(zbuf,
                                acc.at[pl.ds(sid * _NB + q * _ZR, _ZR), :])
            plsc.subcore_barrier()

            def chunk_body(i, _):
                base = sid * _ECT + i * _C
                d1 = pltpu.async_copy(
                    srcoff_hbm.at[pl.ds(chunk * _EP + base, _C)], src_v, sem)
                d2 = pltpu.async_copy(dst_hbm.at[pl.ds(base, _C)], dst_v, sem)
                d3 = pltpu.async_copy(ew_hbm.at[pl.ds(base, _C), :], ewrow_v,
                                      sem)
                d1.wait()
                d2.wait()
                d3.wait()
                pltpu.async_copy(x_hbm.at[src_v], rows_v, sem).wait()

                def edge_body(e, _):
                    for j in range(grp):
                        rows_v[e, pl.ds(j * 16, 16)] = (
                            rows_v[e, pl.ds(j * 16, 16)]
                            * ewrow_v[e, pl.ds(j * 16, 16)])
                    return 0

                lax.fori_loop(0, _C, edge_body, 0)
                pltpu.sync_copy(rows_v, acc.at[dst_v], add=True)
                return 0

            lax.fori_loop(0, nch, chunk_body, 0)
            plsc.subcore_barrier()
            pltpu.sync_copy(
                acc.at[pl.ds(sid * _NB, _NB), :],
                out_hbm.at[pl.ds(chunk * _N + sid * _NB, _NB), :])
            plsc.subcore_barrier()
            return 0

        lax.fori_loop(0, npv, pass_body, 0)

    return hop


def _hop(x8, srcoff, dst, ew, npass):
    return _make_hop()(x8, srcoff, dst, ew,
                       jnp.full((16,), npass, jnp.int32))


_PBLK = 2048


def _prep_body(ea_ref, ewf_ref):
    i = pl.program_id(0)
    ea = ea_ref[...]
    w = jnp.sqrt(ea[:, 0:1] ** 2 + ea[:, 1:2] ** 2 + ea[:, 2:3] ** 2)
    rid = lax.broadcasted_iota(jnp.int32, (_PBLK, 1), 0) + i * _PBLK
    w = jnp.where(rid < _E, w, 0.0)
    ewf_ref[...] = jnp.broadcast_to(w, (_PBLK, _WC))


def _prep(edge_attr):
    return pl.pallas_call(
        _prep_body,
        grid=(_EP // _PBLK,),
        in_specs=[pl.BlockSpec((_PBLK, 3), lambda i: (i, 0))],
        out_specs=pl.BlockSpec((_PBLK, _WC), lambda i: (i, 0)),
        out_shape=jax.ShapeDtypeStruct((_EP, _WC), jnp.float32),
    )(edge_attr)


_RB = 3000


def _dis_body(degm_ref, dis_ref, dis2_ref):
    d = degm_ref[...][:, 0:1]
    good = d > 0
    r = lax.rsqrt(jnp.where(good, d, 1.0))
    dis = jnp.where(good, r, 0.0)
    dis16 = jnp.broadcast_to(dis, (_RB, 16))
    dis_ref[...] = dis16
    dis2_ref[...] = dis16 * dis16


def _dis(degm):
    return pl.pallas_call(
        _dis_body,
        grid=(_N // _RB,),
        in_specs=[pl.BlockSpec((_RB, _WC), lambda i: (i, 0))],
        out_specs=[pl.BlockSpec((_RB, 16), lambda i: (i, 0))] * 2,
        out_shape=[jax.ShapeDtypeStruct((_N, 16), jnp.float32)] * 2,
    )(degm.reshape(_NCK * _N, _WC)[: _N])


_RS = 600


def _scale_call(nc, body, n_m, in_full=True):
    n_s = {_scale_body: 1, _dual_body: 2, _mid_body: 2, _post_body: 1}[body]
    n_out = 2 if body is _dual_body else 1
    in_ck = _NCK if in_full else nc

    def run(*arrays):
        outs = pl.pallas_call(
            body,
            grid=(_N // _RS,),
            in_specs=[pl.BlockSpec((nc, _RS, _WC), lambda i: (0, i, 0))] *
                     n_m + [pl.BlockSpec((_RS, 16), lambda i: (i, 0))] * n_s,
            out_specs=[pl.BlockSpec((nc, _RS, _WC),
                                    lambda i: (0, i, 0))] * n_out,
            out_shape=[jax.ShapeDtypeStruct((_NCK, _N, _WC),
                                            jnp.float32)] * n_out,
        )(*([a.reshape(in_ck, _N, _WC) for a in arrays[:n_m]]
            + list(arrays[n_m:])))
        res = [o.reshape(_NCK * _N, _WC) for o in outs]
        return res[0] if n_out == 1 else res
    return run


def _scale_body(m, s, o):
    o[...] = m[...] * s[...][:, 0:1][None]


def _dual_body(m, s1, s2, o1, o2):
    mv = m[...]
    o1[...] = mv * s1[...][:, 0:1][None]
    o2[...] = mv * s2[...][:, 0:1][None]


def _mid_body(m, init, s2, s1, o):
    o[...] = (m[...] * s2[...][:, 0:1][None]
              + init[...] * s1[...][:, 0:1][None])


def _post_body(m, init, s1, o):
    o[...] = m[...] * s1[...][:, 0:1][None] + init[...]


_R = 480


def _l1_body(h0, h1, h2, h3, w1, w2, b1, dis, h1f, z0, z1, z2, z3):
    hks = [h0, h1, h2, h3]
    zrefs = [z0, z1, z2, z3]
    dcol = dis[...][:, 0:1]
    for b in range(_B):
        c, o = b // 2, (b % 2) * 16
        xb = jnp.concatenate([hk[c, :, o:o + 16] for hk in hks], axis=1)
        pre = jnp.dot(xb, w1[...], preferred_element_type=jnp.float32)
        h1b = jnp.maximum(pre + b1[...], 0.0)
        z2b = jnp.dot(h1b, w2[...], preferred_element_type=jnp.float32)
        h1f[2 * b] = h1b[:, :32]
        h1f[2 * b + 1] = h1b[:, 32:]
        for k in range(4):
            zk = z2b[:, 64 * k:64 * k + 64]
            if k == 3:
                zk = zk * dcol
            zrefs[k][2 * b] = zk[:, :32]
            zrefs[k][2 * b + 1] = zk[:, 32:]


def _l1(hk_list, w1r, w2c, b1r, dis16):
    big = jax.ShapeDtypeStruct((_NCK, _N, _WC), jnp.float32)
    outs = pl.pallas_call(
        _l1_body,
        grid=(_N // _R,),
        in_specs=[pl.BlockSpec((2, _R, _WC), lambda i: (0, i, 0))] * 4 + [
            pl.BlockSpec((64, 64), lambda i: (0, 0)),
            pl.BlockSpec((64, 256), lambda i: (0, 0)),
            pl.BlockSpec((1, 64), lambda i: (0, 0)),
            pl.BlockSpec((_R, 16), lambda i: (i, 0)),
        ],
        out_specs=[pl.BlockSpec((_NCK, _R, _WC), lambda i: (0, i, 0))] * 5,
        out_shape=[big] * 5,
    )(*[h.reshape(-1, _N, _WC)[:2] for h in hk_list], w1r, w2c, b1r, dis16)
    return [o.reshape(_NCK * _N, _WC) for o in outs]


def _l2_body(o2, h1f, w3, b2, dis, z0, z1, z2, z3):
    zrefs = [z0, z1, z2, z3]
    dcol = dis[...][:, 0:1]
    parts = [[] for _ in range(4)]
    for b in range(_B):
        o2b = jnp.concatenate([o2[2 * b], o2[2 * b + 1]], axis=1)
        h1b = jnp.concatenate([h1f[2 * b], h1f[2 * b + 1]], axis=1)
        h2b = (jnp.maximum(o2b + b2[...], 0.0) + h1b) * 0.5
        z3b = jnp.dot(h2b, w3[...], preferred_element_type=jnp.float32)
        for k in range(4):
            parts[k].append(z3b[:, 4 * k:4 * k + 4])
    zeros16 = jnp.zeros((_R, 16), jnp.float32)
    for k in range(4):
        zk = jnp.concatenate(parts[k], axis=1)
        if k == 3:
            zk = zk * dcol
        zrefs[k][0] = jnp.concatenate([zk, zeros16], axis=1)
        zrefs[k][1] = jnp.zeros((_R, _WC), jnp.float32)


def _l2(out2f, h1f, w3c, b2r, dis16):
    small = jax.ShapeDtypeStruct((_NCK, _N, _WC), jnp.float32)
    outs = pl.pallas_call(
        _l2_body,
        grid=(_N // _R,),
        in_specs=[pl.BlockSpec((8, _R, _WC), lambda i: (0, i, 0))] * 2 + [
            pl.BlockSpec((64, 16), lambda i: (0, 0)),
            pl.BlockSpec((1, 64), lambda i: (0, 0)),
            pl.BlockSpec((_R, 16), lambda i: (i, 0)),
        ],
        out_specs=[pl.BlockSpec((2, _R, _WC), lambda i: (0, i, 0))] * 4,
        out_shape=[small] * 4,
    )(out2f.reshape(_NCK, _N, _WC), h1f.reshape(_NCK, _N, _WC),
      w3c, b2r, dis16)
    return [o.reshape(_NCK * _N, _WC) for o in outs]


def _tail_body(lstm_in_ref, b3f_ref, Wih0_ref, Whh0_ref, b0_ref, Wih1_ref,
               Whh1_ref, b1_ref, Wr1_ref, br1_ref, Wr2_ref, br2_ref, Wf_ref,
               bf_ref, We_ref, be_ref, o_ref, eo_ref):
    IN0 = _TGT * _NPT
    xin = lstm_in_ref[...].reshape(_B * _T, IN0)
    xin = jnp.maximum(xin + b3f_ref[...], 0.0)
    xp0 = jnp.dot(xin, Wih0_ref[...], preferred_element_type=jnp.float32)
    xp0 = xp0 + b0_ref[...]
    xp0 = xp0.reshape(_B, _T, 4 * _LH)

    def lstm_cell(gates, c):
        i = jax.nn.sigmoid(gates[:, :_LH])
        f = jax.nn.sigmoid(gates[:, _LH:2 * _LH])
        g = jnp.tanh(gates[:, 2 * _LH:3 * _LH])
        o = jax.nn.sigmoid(gates[:, 3 * _LH:])
        c = f * c + i * g
        h = o * jnp.tanh(c)
        return h, c

    z = jnp.zeros((_B, _LH), jnp.float32)
    h0, c0, h1, c1 = z, z, z, z
    for t in range(_T):
        g0 = xp0[:, t, :] + jnp.dot(h0, Whh0_ref[...],
                                    preferred_element_type=jnp.float32)
        h0, c0 = lstm_cell(g0, c0)
        g1 = (jnp.dot(h0, Wih1_ref[...], preferred_element_type=jnp.float32)
              + jnp.dot(h1, Whh1_ref[...], preferred_element_type=jnp.float32)
              + b1_ref[...])
        h1, c1 = lstm_cell(g1, c1)

    last = jnp.maximum(h1, 0.0)
    r = jnp.maximum(jnp.dot(last, Wr1_ref[...],
                            preferred_element_type=jnp.float32)
                    + br1_ref[...], 0.0)
    r = jnp.dot(r, Wr2_ref[...], preferred_element_type=jnp.float32) + br2_ref[...]
    o_ref[...] = jnp.dot(r, Wf_ref[...],
                         preferred_element_type=jnp.float32) + bf_ref[...]
    eo_ref[...] = jnp.dot(r, We_ref[...],
                          preferred_element_type=jnp.float32) + be_ref[...]


def _tail(lstm_in, b3f, Wih0, Whh0, b0, Wih1, Whh1, b1, Wr1, br1, Wr2, br2,
          Wf, bf, We, be):
    EO = We.shape[1]
    o, eo = pl.pallas_call(
        _tail_body,
        out_shape=[
            jax.ShapeDtypeStruct((_B, 2 * _NPT), jnp.float32),
            jax.ShapeDtypeStruct((_B, EO), jnp.float32),
        ],
    )(lstm_in, b3f, Wih0, Whh0, b0, Wih1, Whh1, b1, Wr1, br1, Wr2, br2,
      Wf, bf, We, be)
    return o, eo


def kernel(x, edge_index, edge_attr, W1, b1, W2, b2, W3, b3, Wih0, Whh0,
           bih0, bhh0, Wih1, Whh1, bih1, bhh1, Wr1, br1, Wr2, br2, Wf1, bf1,
           Wf2, bf2, We, be):
    src = edge_index[0].astype(jnp.int32)
    dst = edge_index[1].astype(jnp.int32)
    pad = _EP - _E
    src_p = jnp.concatenate([src, jnp.zeros((pad,), jnp.int32)])
    dst_p = jnp.concatenate([dst, jnp.zeros((pad,), jnp.int32)])
    off8 = (jnp.arange(_NCK, dtype=jnp.int32) * _N)[:, None]
    srcoff = (src_p[None, :] + off8).reshape(-1)

    ewf = _prep(edge_attr)
    ones8 = jnp.ones((_NCK * _N, _WC), jnp.float32)
    degm = _hop(ones8, srcoff, dst_p, ewf, 1)
    dis16, dis216 = _dis(degm)

    x2 = (x.transpose(1, 0, 2).reshape(_N, _B * _F)
          .reshape(_N, 2, _WC).transpose(1, 0, 2).reshape(2 * _N, _WC))
    scale2 = _scale_call(2, _scale_body, 1, in_full=False)
    dual2 = _scale_call(2, _dual_body, 1)
    mid2 = _scale_call(2, _mid_body, 2)
    post2 = _scale_call(2, _post_body, 2)

    xs = scale2(x2, dis16)
    m = _hop(xs, srcoff, dst_p, ewf, 1)
    ha, has = dual2(m, dis16, dis216)
    m = _hop(has, srcoff, dst_p, ewf, 1)
    hb, hbs = dual2(m, dis16, dis216)
    m = _hop(hbs, srcoff, dst_p, ewf, 1)
    hc = _scale_call(2, _scale_body, 1)(m, dis16)

    w1r = W1.reshape(4 * _F, _H)
    w2c = W2.transpose(1, 0, 2).reshape(_H, 4 * _H)
    b1r = b1.reshape(1, _H)
    h1f, z2_0, z2_1, z2_2, z2_3s = _l1([x2, ha, hb, hc], w1r, w2c, b1r, dis16)

    mid8 = _scale_call(8, _mid_body, 2)
    post8 = _scale_call(8, _post_body, 2)
    m = _hop(z2_3s, srcoff, dst_p, ewf, 4)
    t = mid8(m, z2_2, dis216, dis16)
    m = _hop(t, srcoff, dst_p, ewf, 4)
    t = mid8(m, z2_1, dis216, dis16)
    m = _hop(t, srcoff, dst_p, ewf, 4)
    out2 = post8(m, z2_0, dis16)

    w3c = W3.transpose(1, 0, 2).reshape(_H, 4 * _TGT)
    b2r = b2.reshape(1, _H)
    z3_0, z3_1, z3_2, z3_3s = _l2(out2, h1f, w3c, b2r, dis16)

    m = _hop(z3_3s, srcoff, dst_p, ewf, 1)
    t = mid2(m, z3_2, dis216, dis16)
    m = _hop(t, srcoff, dst_p, ewf, 1)
    t = mid2(m, z3_1, dis216, dis16)
    m = _hop(t, srcoff, dst_p, ewf, 1)
    out3 = post2(m, z3_0, dis16)

    lstm_in = (out3[:_N, :16].reshape(_T, _NPT, _B, _TGT)
               .transpose(2, 0, 1, 3).reshape(_B, _T, _NPT * _TGT))
    b3f = jnp.tile(b3, _NPT).reshape(1, _NPT * _TGT)

    Wf = jnp.concatenate([Wf1, Wf2], axis=1)
    bf = jnp.concatenate([bf1, bf2], axis=0)
    o, eo = _tail(lstm_in, b3f, Wih0, Whh0, bih0 + bhh0, Wih1, Whh1,
                  bih1 + bhh1, Wr1, br1, Wr2, br2, Wf, bf, We, be)
    o = o.reshape(_B, 2, _NPT)
    eo = eo.reshape(_B, 2, 7064)
    return o, eo
